# baseline (speedup 1.0000x reference)
from functools import partial

import jax
import jax.numpy as jnp
from jax.experimental import pallas as pl
from jax.experimental.pallas import tpu as pltpu

_KSZ = 3
_HALO = 64


def _conv_body(x_ref, w_ref, b_ref, o_ref, xp_ref, xs_ref, acc_ref,
               *, nb, l, wsp, cp):
    li = jax.lax.broadcasted_iota(jnp.int32, (l, 1), 0) % wsp
    not_left, not_right = li != 0, li != wsp - 1

    xp_ref[0:_HALO, :] = jnp.zeros((_HALO, cp), jnp.bfloat16)
    xp_ref[_HALO + l:, :] = jnp.zeros((_HALO, cp), jnp.bfloat16)

    for bi in range(nb):
        xp_ref[_HALO:_HALO + l, :] = x_ref[bi].astype(jnp.bfloat16)
        for ki in range(_KSZ):
            for kj in range(_KSZ):
                t = ki * _KSZ + kj
                st = _HALO + (ki - 1) * wsp + (kj - 1)
                val = xp_ref[st:st + l, :]
                if kj == 0:
                    val = jnp.where(not_left, val, jnp.bfloat16(0))
                elif kj == _KSZ - 1:
                    val = jnp.where(not_right, val, jnp.bfloat16(0))
                xs_ref[:, t * cp:(t + 1) * cp] = val
        acc = jnp.dot(xs_ref[...], w_ref[...],
                      preferred_element_type=jnp.float32) + b_ref[...]
        nn = acc.shape[1] // 2
        r = l // nn
        acc_ref[...] = acc.reshape(nn, r, 2 * nn)
        for jlo in range(r):
            for g2 in range(2):
                o_ref[bi, jlo * nn:(jlo + 1) * nn,
                      g2 * nn:(g2 + 1) * nn] = (
                    acc_ref[:, jlo, g2 * nn:(g2 + 1) * nn].T)


def kernel(x, w, b):
    B, C, H, W = x.shape
    G, N, K = w.shape
    cg = C // G
    gc = G // 2
    cp = 2 * cg
    l = H * W

    xh = x.transpose(0, 2, 3, 1).reshape(B, l, C)

    wt = w.reshape(G, N, cg, _KSZ * _KSZ).transpose(0, 3, 2, 1)
    wt = wt.reshape(gc, 2, _KSZ * _KSZ, cg, N)
    eye = jnp.eye(2, dtype=wt.dtype)
    wp = jnp.einsum("chtkn,hj->cthkjn", wt, eye)
    wp = wp.reshape(gc, _KSZ * _KSZ * cp, 2 * N).astype(jnp.bfloat16)
    bp = b.reshape(gc, 1, 2 * N).astype(jnp.float32)

    nb = 4 if B % 4 == 0 else 1
    body = partial(_conv_body, nb=nb, l=l, wsp=W, cp=cp)

    out = pl.pallas_call(
        body,
        out_shape=jax.ShapeDtypeStruct((B, l, G * N), jnp.float32),
        grid_spec=pltpu.PrefetchScalarGridSpec(
            num_scalar_prefetch=0,
            grid=(gc, B // nb),
            in_specs=[
                pl.BlockSpec((nb, l, cp), lambda c, bb: (bb, 0, c)),
                pl.BlockSpec((pl.Squeezed(), _KSZ * _KSZ * cp, 2 * N),
                             lambda c, bb: (c, 0, 0)),
                pl.BlockSpec((pl.Squeezed(), 1, 2 * N),
                             lambda c, bb: (c, 0, 0)),
            ],
            out_specs=pl.BlockSpec((nb, l, 2 * N),
                                   lambda c, bb: (bb, 0, c)),
            scratch_shapes=[
                pltpu.VMEM((l + 2 * _HALO, cp), jnp.bfloat16),
                pltpu.VMEM((l, _KSZ * _KSZ * cp), jnp.bfloat16),
                pltpu.VMEM((N, l // N, 2 * N), jnp.float32),
            ],
        ),
        compiler_params=pltpu.CompilerParams(
            dimension_semantics=("parallel", "parallel"),
            vmem_limit_bytes=48 * 1024 * 1024,
        ),
    )(xh, wp, bp)

    return out.reshape(B, H, W, G * N).transpose(0, 3, 1, 2)


# speedup vs baseline: 1.0258x; 1.0258x over previous
from functools import partial

import jax
import jax.numpy as jnp
from jax.experimental import pallas as pl
from jax.experimental.pallas import tpu as pltpu

_KSZ = 3
_HALO = 64


def _conv_body(x_ref, w_ref, b_ref, o_ref, xp_ref, xs_ref, acc_ref,
               *, nb, gc, l, wsp, cp):
    li = jax.lax.broadcasted_iota(jnp.int32, (l, 1), 0) % wsp
    not_left, not_right = li != 0, li != wsp - 1

    xp_ref[0:_HALO, :] = jnp.zeros((_HALO, cp), jnp.bfloat16)
    xp_ref[_HALO + l:, :] = jnp.zeros((_HALO, cp), jnp.bfloat16)

    for bi in range(nb):
        for c in range(gc):
            xp_ref[_HALO:_HALO + l, :] = (
                x_ref[bi, :, c * cp:(c + 1) * cp].astype(jnp.bfloat16))
            for ki in range(_KSZ):
                for kj in range(_KSZ):
                    t = ki * _KSZ + kj
                    st = _HALO + (ki - 1) * wsp + (kj - 1)
                    val = xp_ref[st:st + l, :]
                    if kj == 0:
                        val = jnp.where(not_left, val, jnp.bfloat16(0))
                    elif kj == _KSZ - 1:
                        val = jnp.where(not_right, val, jnp.bfloat16(0))
                    xs_ref[:, t * cp:(t + 1) * cp] = val
            acc = jnp.dot(xs_ref[...], w_ref[c],
                          preferred_element_type=jnp.float32) + b_ref[c]
            nn = acc.shape[1] // 2
            r = l // nn
            acc_ref[...] = acc.reshape(nn, r, 2 * nn)
            for jlo in range(r):
                for g2 in range(2):
                    o_ref[bi, jlo * nn:(jlo + 1) * nn,
                          (c * 2 + g2) * nn:(c * 2 + g2 + 1) * nn] = (
                        acc_ref[:, jlo, g2 * nn:(g2 + 1) * nn].T)


def kernel(x, w, b):
    B, C, H, W = x.shape
    G, N, K = w.shape
    cg = C // G
    gc = G // 2
    cp = 2 * cg
    l = H * W

    xh = x.transpose(0, 2, 3, 1).reshape(B, l, C)

    wt = w.reshape(G, N, cg, _KSZ * _KSZ).transpose(0, 3, 2, 1)
    wt = wt.reshape(gc, 2, _KSZ * _KSZ, cg, N)
    eye = jnp.eye(2, dtype=wt.dtype)
    wp = jnp.einsum("chtkn,hj->cthkjn", wt, eye)
    wp = wp.reshape(gc, _KSZ * _KSZ * cp, 2 * N).astype(jnp.bfloat16)
    bp = b.reshape(gc, 1, 2 * N).astype(jnp.float32)

    nb = 4 if B % 4 == 0 else 1
    body = partial(_conv_body, nb=nb, gc=gc, l=l, wsp=W, cp=cp)

    out = pl.pallas_call(
        body,
        out_shape=jax.ShapeDtypeStruct((B, l, G * N), jnp.float32),
        grid_spec=pltpu.PrefetchScalarGridSpec(
            num_scalar_prefetch=0,
            grid=(B // nb,),
            in_specs=[
                pl.BlockSpec((nb, l, C), lambda bb: (bb, 0, 0)),
                pl.BlockSpec((gc, _KSZ * _KSZ * cp, 2 * N),
                             lambda bb: (0, 0, 0)),
                pl.BlockSpec((gc, 1, 2 * N), lambda bb: (0, 0, 0)),
            ],
            out_specs=pl.BlockSpec((nb, l, G * N), lambda bb: (bb, 0, 0)),
            scratch_shapes=[
                pltpu.VMEM((l + 2 * _HALO, cp), jnp.bfloat16),
                pltpu.VMEM((l, _KSZ * _KSZ * cp), jnp.bfloat16),
                pltpu.VMEM((N, l // N, 2 * N), jnp.float32),
            ],
        ),
        compiler_params=pltpu.CompilerParams(
            dimension_semantics=("parallel",),
            vmem_limit_bytes=48 * 1024 * 1024,
        ),
    )(xh, wp, bp)

    return out.reshape(B, H, W, G * N).transpose(0, 3, 1, 2)


# speedup vs baseline: 1.0307x; 1.0048x over previous
from functools import partial

import jax
import jax.numpy as jnp
from jax.experimental import pallas as pl
from jax.experimental.pallas import tpu as pltpu

_KSZ = 3
_HALO = 64


def _conv_body(x_ref, w_ref, b_ref, o_ref, xp_ref, xs_ref, acc_ref,
               *, nb, gc, l, wsp, cp):
    li = jax.lax.broadcasted_iota(jnp.int32, (l, 1), 0) % wsp
    not_left, not_right = li != 0, li != wsp - 1

    xp_ref[0:_HALO, :] = jnp.zeros((_HALO, cp), jnp.bfloat16)
    xp_ref[_HALO + l:, :] = jnp.zeros((_HALO, cp), jnp.bfloat16)

    for bi in range(nb):
        for c in range(gc):
            xp_ref[_HALO:_HALO + l, :] = (
                x_ref[bi, :, c * cp:(c + 1) * cp].astype(jnp.bfloat16))
            for ki in range(_KSZ):
                for kj in range(_KSZ):
                    t = ki * _KSZ + kj
                    st = _HALO + (ki - 1) * wsp + (kj - 1)
                    val = xp_ref[st:st + l, :]
                    if kj == 0:
                        val = jnp.where(not_left, val, jnp.bfloat16(0))
                    elif kj == _KSZ - 1:
                        val = jnp.where(not_right, val, jnp.bfloat16(0))
                    xs_ref[:, t * cp:(t + 1) * cp] = val
            acc = jnp.dot(xs_ref[...], w_ref[c],
                          preferred_element_type=jnp.float32) + b_ref[c]
            nn = acc.shape[1] // 2
            r = l // nn
            acc_ref[...] = acc.reshape(nn, r, 2 * nn)
            for jlo in range(r):
                blk_t = acc_ref[:, jlo, :].T
                for g2 in range(2):
                    o_ref[bi, jlo * nn:(jlo + 1) * nn,
                          (c * 2 + g2) * nn:(c * 2 + g2 + 1) * nn] = (
                        blk_t[g2 * nn:(g2 + 1) * nn, :])


def kernel(x, w, b):
    B, C, H, W = x.shape
    G, N, K = w.shape
    cg = C // G
    gc = G // 2
    cp = 2 * cg
    l = H * W

    xh = x.transpose(0, 2, 3, 1).reshape(B, l, C)

    wt = w.reshape(G, N, cg, _KSZ * _KSZ).transpose(0, 3, 2, 1)
    wt = wt.reshape(gc, 2, _KSZ * _KSZ, cg, N)
    eye = jnp.eye(2, dtype=wt.dtype)
    wp = jnp.einsum("chtkn,hj->cthkjn", wt, eye)
    wp = wp.reshape(gc, _KSZ * _KSZ * cp, 2 * N).astype(jnp.bfloat16)
    bp = b.reshape(gc, 1, 2 * N).astype(jnp.float32)

    nb = 4 if B % 4 == 0 else 1
    body = partial(_conv_body, nb=nb, gc=gc, l=l, wsp=W, cp=cp)

    out = pl.pallas_call(
        body,
        out_shape=jax.ShapeDtypeStruct((B, l, G * N), jnp.float32),
        grid_spec=pltpu.PrefetchScalarGridSpec(
            num_scalar_prefetch=0,
            grid=(B // nb,),
            in_specs=[
                pl.BlockSpec((nb, l, C), lambda bb: (bb, 0, 0)),
                pl.BlockSpec((gc, _KSZ * _KSZ * cp, 2 * N),
                             lambda bb: (0, 0, 0)),
                pl.BlockSpec((gc, 1, 2 * N), lambda bb: (0, 0, 0)),
            ],
            out_specs=pl.BlockSpec((nb, l, G * N), lambda bb: (bb, 0, 0)),
            scratch_shapes=[
                pltpu.VMEM((l + 2 * _HALO, cp), jnp.bfloat16),
                pltpu.VMEM((l, _KSZ * _KSZ * cp), jnp.bfloat16),
                pltpu.VMEM((N, l // N, 2 * N), jnp.float32),
            ],
        ),
        compiler_params=pltpu.CompilerParams(
            dimension_semantics=("parallel",),
            vmem_limit_bytes=48 * 1024 * 1024,
        ),
    )(xh, wp, bp)

    return out.reshape(B, H, W, G * N).transpose(0, 3, 1, 2)
